# baseline (speedup 1.0000x reference)
import functools

import jax
import jax.numpy as jnp
from jax import lax
from jax.experimental import pallas as pl
from jax.experimental.pallas import tpu as pltpu
from jax.experimental.pallas import tpu_sc as plsc

N = 10000
NP = 10240
E = 320000
D = 128
ALPHA = 0.1
KHOP = 4

NC = 2
NS = 16
NW = NC * NS
L = 16

SP = NP // NS
SPW = NP // NW
CC = 32

EW = E // NW

E16 = E // NS
CPW = 80
NCHW = EW // CPW

_MESH = plsc.VectorSubcoreMesh(
    core_axis_name="c", subcore_axis_name="s", num_cores=NC, num_subcores=NS)

_f32 = jnp.float32

_SC_PARAMS = pltpu.CompilerParams(needs_layout_passes=False)

_LANE_MASKS = None


def _lane_masks():
    lane = lax.iota(jnp.int32, L)
    return [lane == l for l in range(L)]


@functools.partial(
    pl.kernel,
    out_type=(jax.ShapeDtypeStruct((NW, 1, NP), _f32),
              jax.ShapeDtypeStruct((NW, 1, NP), _f32)),
    mesh=_MESH,
    compiler_params=_SC_PARAMS,
    scratch_types=[
        pltpu.VMEM((EW,), jnp.int32),
        pltpu.VMEM((EW,), jnp.int32),
        pltpu.VMEM((1, NP), _f32),
        pltpu.VMEM((1, NP), _f32),
    ],
)
def _deg_kernel(src_flat, dst_flat, zrow, outs, outd,
                src_v, dst_v, hs, hd):
    c = lax.axis_index("c")
    s = lax.axis_index("s")
    w = c * NS + s
    pltpu.sync_copy(src_flat.at[pl.ds(w * EW, EW)], src_v)
    pltpu.sync_copy(dst_flat.at[pl.ds(w * EW, EW)], dst_v)
    pltpu.sync_copy(zrow, hs)
    pltpu.sync_copy(zrow, hd)

    masks = _lane_masks()
    z16 = jnp.zeros((L,), jnp.int32)
    ones = jnp.ones((L,), _f32)

    @pl.loop(0, EW // L)
    def _(i):
        si = src_v[pl.ds(i * L, L)]
        di = dst_v[pl.ds(i * L, L)]
        plsc.addupdate_scatter(hs, [z16, si], ones)
        plsc.addupdate_scatter(hd, [z16, di], ones)

    pltpu.sync_copy(hs, outs.at[w])
    pltpu.sync_copy(hd, outd.at[w])


@functools.partial(
    pl.kernel,
    out_type=jax.ShapeDtypeStruct((NC, NP, D), _f32),
    mesh=_MESH,
    compiler_params=_SC_PARAMS,
    scratch_types=[
        pltpu.VMEM((8, 2, CPW), jnp.int32),
        pltpu.VMEM((4, CPW, D), _f32),
        pltpu.VMEM_SHARED((NP, D), _f32),
        pltpu.SemaphoreType.DMA,
        pltpu.SemaphoreType.DMA,
        pltpu.SemaphoreType.DMA,
    ],
)
def _prop_kernel(f_hbm, idxm, zrows, out,
                 idxb, rows, acc_sh, gsem, isem, ssem):
    c = lax.axis_index("c")
    s = lax.axis_index("s")
    w = c * NS + s
    row0 = s * SP

    pltpu.sync_copy(zrows, acc_sh.at[pl.ds(row0, SP)])
    plsc.subcore_barrier()

    pltpu.sync_copy(idxm.at[w, 0], idxb.at[0])
    pltpu.sync_copy(idxm.at[w, 1], idxb.at[1])
    pltpu.async_copy(f_hbm.at[idxb.at[0, 0]], rows.at[0], gsem)
    pltpu.async_copy(f_hbm.at[idxb.at[1, 0]], rows.at[1], gsem)
    pltpu.async_copy(idxm.at[w, 2], idxb.at[2], isem)

    def slot(ch, m4, m8):
        mg4 = (m4 + 2) % 4
        mg8 = (m8 + 2) % 8
        pltpu.make_async_copy(
            f_hbm.at[idxb.at[m8, 0]], rows.at[m4], gsem).wait()
        pltpu.async_copy(rows.at[m4], acc_sh.at[idxb.at[m8, 1]], ssem,
                         add=True)

        @pl.when(ch >= 1)
        def _():
            pltpu.make_async_copy(
                rows.at[(m4 + 3) % 4], acc_sh.at[idxb.at[m8, 1]],
                ssem).wait()

        @pl.when(ch + 2 < NCHW)
        def _():
            pltpu.make_async_copy(
                idxm.at[w, ch + 2], idxb.at[mg8], isem).wait()
            pltpu.async_copy(f_hbm.at[idxb.at[mg8, 0]], rows.at[mg4], gsem)

        @pl.when(ch + 3 < NCHW)
        def _():
            pltpu.async_copy(idxm.at[w, ch + 3], idxb.at[(m8 + 3) % 8], isem)

    @pl.loop(0, NCHW - 5, step=8)
    def _(base):
        for j in range(8):
            slot(base + j, j % 4, j)

    for j in range(5):
        slot(NCHW - 5 + j, j % 4, j)

    pltpu.make_async_copy(rows.at[0], acc_sh.at[idxb.at[0, 1]], ssem).wait()
    plsc.subcore_barrier()
    pltpu.sync_copy(acc_sh.at[pl.ds(row0, SP)],
                    out.at[c, pl.ds(row0, SP)])


def _edge_w(es_v, ed_v, si, di):
    sv = plsc.load_gather(es_v, [si])
    dv = plsc.load_gather(ed_v, [di])
    x = sv + dv
    z = jnp.exp(x + x)
    t = 1.0 - 2.0 / (z + 1.0)
    return jnp.exp(t)


@functools.partial(
    pl.kernel,
    out_type=jax.ShapeDtypeStruct((E,), _f32),
    mesh=_MESH,
    compiler_params=_SC_PARAMS,
    scratch_types=[
        pltpu.VMEM((N,), _f32),
        pltpu.VMEM((N,), _f32),
        pltpu.VMEM((EW,), jnp.int32),
        pltpu.VMEM((EW,), jnp.int32),
        pltpu.VMEM((EW,), jnp.int32),
        pltpu.VMEM((EW,), jnp.int32),
        pltpu.VMEM((EW,), _f32),
        pltpu.VMEM((1, NP), _f32),
        pltpu.VMEM((SP,), _f32),
        pltpu.VMEM((SP,), _f32),
        pltpu.VMEM((NP,), _f32),
        pltpu.VMEM((EW,), _f32),
        pltpu.VMEM_SHARED((NS, 1, NP), _f32),
        pltpu.VMEM_SHARED((1, NP), _f32),
    ],
)
def _att_kernel(es_hbm, ed_hbm, src_flat, dst_flat, zrow, out,
                es_v, ed_v, src_a, dst_a, src_b, dst_b, w_v,
                hist, tmp_v, dacc_v, denom_v, att_v, stage_sh, den_sh):
    c = lax.axis_index("c")
    s = lax.axis_index("s")
    w = c * NS + s
    wm = (NC - 1 - c) * NS + s

    pltpu.sync_copy(es_hbm, es_v)
    pltpu.sync_copy(ed_hbm, ed_v)
    pltpu.sync_copy(src_flat.at[pl.ds(w * EW, EW)], src_a)
    pltpu.sync_copy(dst_flat.at[pl.ds(w * EW, EW)], dst_a)
    pltpu.sync_copy(src_flat.at[pl.ds(wm * EW, EW)], src_b)
    pltpu.sync_copy(dst_flat.at[pl.ds(wm * EW, EW)], dst_b)
    pltpu.sync_copy(zrow, hist)

    z16 = jnp.zeros((L,), jnp.int32)

    def p1a(sl):
        wv = _edge_w(es_v, ed_v, src_a[sl], dst_a[sl])
        w_v[sl] = wv
        plsc.addupdate_scatter(hist, [z16, dst_a[sl]], wv)

    def p1b(sl):
        wv = _edge_w(es_v, ed_v, src_b[sl], dst_b[sl])
        plsc.addupdate_scatter(hist, [z16, dst_b[sl]], wv)

    @pl.loop(0, EW // L - 1, step=2)
    def _(i):
        for u in range(2):
            p1a(pl.ds((i + u) * L, L))

    p1a(pl.ds(EW - L, L))

    @pl.loop(0, EW // L - 1, step=2)
    def _(i):
        for u in range(2):
            p1b(pl.ds((i + u) * L, L))

    p1b(pl.ds(EW - L, L))

    pltpu.sync_copy(hist, stage_sh.at[s])
    plsc.subcore_barrier()

    @pl.loop(0, SP // L)
    def _(i):
        dacc_v[pl.ds(i * L, L)] = jnp.zeros((L,), _f32)

    for t in range(NS):
        pltpu.sync_copy(stage_sh.at[t, 0, pl.ds(s * SP, SP)], tmp_v)

        @pl.loop(0, SP // L)
        def _(i):
            sl = pl.ds(i * L, L)
            dacc_v[sl] = dacc_v[sl] + tmp_v[sl]

    pltpu.sync_copy(dacc_v, den_sh.at[0, pl.ds(s * SP, SP)])
    plsc.subcore_barrier()
    pltpu.sync_copy(den_sh.at[0], denom_v)

    def p2(sl):
        dn = plsc.load_gather(denom_v, [dst_a[sl]])
        att_v[sl] = w_v[sl] / dn

    @pl.loop(0, EW // L - 1, step=2)
    def _(i):
        for u in range(2):
            p2(pl.ds((i + u) * L, L))

    p2(pl.ds(EW - L, L))

    pltpu.sync_copy(att_v, out.at[pl.ds(w * EW, EW)])


BLK = 2000
GRID = N // BLK
BLKP = 2048
GRIDP = NP // BLKP


def _row_spec(width, block=BLK):
    return pl.BlockSpec((block, width), lambda i: (i, 0))


def _full_spec(shape):
    return pl.BlockSpec(shape, lambda i: tuple(0 for _ in shape))


def _mlp_body(h_ref, w1_ref, b1_ref, w2_ref, b2_ref, o_ref):
    x = jnp.dot(h_ref[...], w1_ref[...], preferred_element_type=_f32)
    x = jnp.maximum(x + b1_ref[...][None, :], 0.0)
    x = jnp.dot(x, w2_ref[...], preferred_element_type=_f32)
    o_ref[...] = x + b2_ref[...][None, :]


def _mlp(h, W1, b1, W2, b2):
    return pl.pallas_call(
        _mlp_body,
        grid=(GRID,),
        in_specs=[_row_spec(D), _full_spec((D, D)), _full_spec((D,)),
                  _full_spec((D, D)), _full_spec((D,))],
        out_specs=_row_spec(D),
        out_shape=jax.ShapeDtypeStruct((NP, D), _f32),
    )(h, W1, b1, W2, b2)


def _prep_body(hs_ref, hd_ref, x_ref, a_ref, b_ref, nd_ref, fs0_ref):
    degs = jnp.sum(hs_ref[...][:, 0, :], axis=0)[:, None]
    degd = jnp.sum(hd_ref[...][:, 0, :], axis=0)[:, None]
    ns = jax.lax.rsqrt(jnp.maximum(degs, 1.0))
    nd = jax.lax.rsqrt(jnp.maximum(degd, 1.0))
    a_ref[...] = (1.0 - ALPHA) * ns * nd
    b_ref[...] = ALPHA * ns
    nd_ref[...] = nd
    fs0_ref[...] = x_ref[...] * ns


def _prep(hS, hD, x):
    return pl.pallas_call(
        _prep_body,
        grid=(GRIDP,),
        in_specs=[pl.BlockSpec((NW, 1, BLKP), lambda i: (0, 0, i)),
                  pl.BlockSpec((NW, 1, BLKP), lambda i: (0, 0, i)),
                  _row_spec(D, BLKP)],
        out_specs=(_row_spec(1, BLKP), _row_spec(1, BLKP),
                   _row_spec(1, BLKP), _row_spec(D, BLKP)),
        out_shape=(jax.ShapeDtypeStruct((NP, 1), _f32),
                   jax.ShapeDtypeStruct((NP, 1), _f32),
                   jax.ShapeDtypeStruct((NP, 1), _f32),
                   jax.ShapeDtypeStruct((NP, D), _f32)),
    )(hS, hD, x)


def _combine_body(p0, p1, x0, a_ref, b_ref, fs_ref):
    agg = p0[...][0] + p1[...][0]
    fs_ref[...] = a_ref[...] * agg + b_ref[...] * x0[...]


def _combine(acc, x0, A, B):
    return pl.pallas_call(
        _combine_body,
        grid=(GRIDP,),
        in_specs=[pl.BlockSpec((1, BLKP, D), lambda i: (0, i, 0)),
                  pl.BlockSpec((1, BLKP, D), lambda i: (1, i, 0)),
                  _row_spec(D, BLKP), _row_spec(1, BLKP),
                  _row_spec(1, BLKP)],
        out_specs=_row_spec(D, BLKP),
        out_shape=jax.ShapeDtypeStruct((NP, D), _f32),
    )(acc, acc, x0, A, B)


def _final_body(p0, p1, x0, nd, wsrc, wdst, logp_ref, es_ref, ed_ref):
    agg = p0[...][0] + p1[...][0]
    feat = (1.0 - ALPHA) * (agg * nd[...]) + ALPHA * x0[...]
    m = jnp.max(feat, axis=1, keepdims=True)
    lse = jnp.log(jnp.sum(jnp.exp(feat - m), axis=1, keepdims=True)) + m
    logp_ref[...] = feat - lse
    es_ref[...] = jnp.sum(feat * wsrc[...][:, 0][None, :], axis=1,
                          keepdims=True)
    ed_ref[...] = jnp.sum(feat * wdst[...][:, 0][None, :], axis=1,
                          keepdims=True)


def _final(acc, x0, nd, Wsrc, Wdst):
    return pl.pallas_call(
        _final_body,
        grid=(GRID,),
        in_specs=[pl.BlockSpec((1, BLK, D), lambda i: (0, i, 0)),
                  pl.BlockSpec((1, BLK, D), lambda i: (1, i, 0)),
                  _row_spec(D), _row_spec(1),
                  _full_spec((D, 1)), _full_spec((D, 1))],
        out_specs=(_row_spec(D), _row_spec(1), _row_spec(1)),
        out_shape=(jax.ShapeDtypeStruct((N, D), _f32),
                   jax.ShapeDtypeStruct((N, 1), _f32),
                   jax.ShapeDtypeStruct((N, 1), _f32)),
    )(acc, acc, x0, nd, Wsrc, Wdst)


def kernel(h, edge_index, W1, b1, W2, b2, Wsrc, Wdst):
    src = edge_index[0]
    dst = edge_index[1]
    idxm = jnp.stack([src.reshape(NW, NCHW, CPW),
                      dst.reshape(NW, NCHW, CPW)], axis=2)

    zrow = jnp.zeros((1, NP), _f32)
    zrows = jnp.zeros((SP, D), _f32)

    x = _mlp(h, W1, b1, W2, b2)
    hS, hD = _deg_kernel(src, dst, zrow)
    A, B, nd, fs0 = _prep(hS, hD, x)
    fs = fs0
    for _ in range(KHOP - 1):
        acc = _prop_kernel(fs, idxm, zrows)
        fs = _combine(acc, x, A, B)
    acc = _prop_kernel(fs, idxm, zrows)
    logp, es, ed = _final(acc, x, nd, Wsrc, Wdst)
    att = _att_kernel(es.reshape(N), ed.reshape(N), src, dst, zrow)
    return logp, att.reshape(E, 1)


# speedup vs baseline: 1.0210x; 1.0210x over previous
import functools

import jax
import jax.numpy as jnp
from jax import lax
from jax.experimental import pallas as pl
from jax.experimental.pallas import tpu as pltpu
from jax.experimental.pallas import tpu_sc as plsc

N = 10000
NP = 10240
E = 320000
D = 128
ALPHA = 0.1
KHOP = 4

NC = 2
NS = 16
NW = NC * NS
L = 16

SP = NP // NS
SPW = NP // NW
CC = 32

EW = E // NW

E16 = E // NS
CPW = 80
NCHW = EW // CPW

_MESH = plsc.VectorSubcoreMesh(
    core_axis_name="c", subcore_axis_name="s", num_cores=NC, num_subcores=NS)

_f32 = jnp.float32

_SC_PARAMS = pltpu.CompilerParams(needs_layout_passes=False)

_LANE_MASKS = None


def _lane_masks():
    lane = lax.iota(jnp.int32, L)
    return [lane == l for l in range(L)]


@functools.partial(
    pl.kernel,
    out_type=(jax.ShapeDtypeStruct((NW, 1, NP), _f32),
              jax.ShapeDtypeStruct((NW, 1, NP), _f32)),
    mesh=_MESH,
    compiler_params=_SC_PARAMS,
    scratch_types=[
        pltpu.VMEM((EW,), jnp.int32),
        pltpu.VMEM((EW,), jnp.int32),
        pltpu.VMEM((1, NP), _f32),
        pltpu.VMEM((1, NP), _f32),
    ],
)
def _deg_kernel(src_flat, dst_flat, zrow, outs, outd,
                src_v, dst_v, hs, hd):
    c = lax.axis_index("c")
    s = lax.axis_index("s")
    w = c * NS + s
    pltpu.sync_copy(src_flat.at[pl.ds(w * EW, EW)], src_v)
    pltpu.sync_copy(dst_flat.at[pl.ds(w * EW, EW)], dst_v)
    pltpu.sync_copy(zrow, hs)
    pltpu.sync_copy(zrow, hd)

    masks = _lane_masks()
    z16 = jnp.zeros((L,), jnp.int32)
    ones = jnp.ones((L,), _f32)

    @pl.loop(0, EW // L)
    def _(i):
        si = src_v[pl.ds(i * L, L)]
        di = dst_v[pl.ds(i * L, L)]
        plsc.addupdate_scatter(hs, [z16, si], ones)
        plsc.addupdate_scatter(hd, [z16, di], ones)

    pltpu.sync_copy(hs, outs.at[w])
    pltpu.sync_copy(hd, outd.at[w])


@functools.partial(
    pl.kernel,
    out_type=jax.ShapeDtypeStruct((NC, NP, D), _f32),
    mesh=_MESH,
    compiler_params=_SC_PARAMS,
    scratch_types=[
        pltpu.VMEM((8, 2, CPW), jnp.int32),
        pltpu.VMEM((4, CPW, D), _f32),
        pltpu.VMEM_SHARED((NP, D), _f32),
        pltpu.SemaphoreType.DMA,
        pltpu.SemaphoreType.DMA,
        pltpu.SemaphoreType.DMA,
    ],
)
def _prop_kernel(f_hbm, idxm, zrows, out,
                 idxb, rows, acc_sh, gsem, isem, ssem):
    c = lax.axis_index("c")
    s = lax.axis_index("s")
    w = c * NS + s
    row0 = s * SP

    pltpu.sync_copy(zrows, acc_sh.at[pl.ds(row0, SP)])
    plsc.subcore_barrier()

    pltpu.sync_copy(idxm.at[w, 0], idxb.at[0])
    pltpu.sync_copy(idxm.at[w, 1], idxb.at[1])
    pltpu.sync_copy(idxm.at[w, 2], idxb.at[2])
    pltpu.async_copy(f_hbm.at[idxb.at[0, 0]], rows.at[0], gsem)
    pltpu.async_copy(f_hbm.at[idxb.at[1, 0]], rows.at[1], gsem)
    pltpu.async_copy(f_hbm.at[idxb.at[2, 0]], rows.at[2], gsem)
    pltpu.async_copy(idxm.at[w, 3], idxb.at[3], isem)

    def slot(ch, m4, m8):
        mg4 = (m4 + 3) % 4
        mg8 = (m8 + 3) % 8
        pltpu.make_async_copy(
            f_hbm.at[idxb.at[m8, 0]], rows.at[m4], gsem).wait()
        pltpu.async_copy(rows.at[m4], acc_sh.at[idxb.at[m8, 1]], ssem,
                         add=True)

        @pl.when(ch >= 1)
        def _():
            pltpu.make_async_copy(
                rows.at[mg4], acc_sh.at[idxb.at[m8, 1]],
                ssem).wait()

        @pl.when(ch + 3 < NCHW)
        def _():
            pltpu.make_async_copy(
                idxm.at[w, ch + 3], idxb.at[mg8], isem).wait()
            pltpu.async_copy(f_hbm.at[idxb.at[mg8, 0]], rows.at[mg4], gsem)

        @pl.when(ch + 4 < NCHW)
        def _():
            pltpu.async_copy(idxm.at[w, ch + 4], idxb.at[(m8 + 4) % 8], isem)

    @pl.loop(0, NCHW - 5, step=8)
    def _(base):
        for j in range(8):
            slot(base + j, j % 4, j)

    for j in range(5):
        slot(NCHW - 5 + j, j % 4, j)

    pltpu.make_async_copy(rows.at[0], acc_sh.at[idxb.at[0, 1]], ssem).wait()
    plsc.subcore_barrier()
    pltpu.sync_copy(acc_sh.at[pl.ds(row0, SP)],
                    out.at[c, pl.ds(row0, SP)])


def _edge_w(es_v, ed_v, si, di):
    sv = plsc.load_gather(es_v, [si])
    dv = plsc.load_gather(ed_v, [di])
    x = sv + dv
    z = jnp.exp(x + x)
    t = 1.0 - 2.0 / (z + 1.0)
    return jnp.exp(t)


@functools.partial(
    pl.kernel,
    out_type=jax.ShapeDtypeStruct((E,), _f32),
    mesh=_MESH,
    compiler_params=_SC_PARAMS,
    scratch_types=[
        pltpu.VMEM((N,), _f32),
        pltpu.VMEM((N,), _f32),
        pltpu.VMEM((EW,), jnp.int32),
        pltpu.VMEM((EW,), jnp.int32),
        pltpu.VMEM((EW,), jnp.int32),
        pltpu.VMEM((EW,), jnp.int32),
        pltpu.VMEM((EW,), _f32),
        pltpu.VMEM((1, NP), _f32),
        pltpu.VMEM((SP,), _f32),
        pltpu.VMEM((SP,), _f32),
        pltpu.VMEM((NP,), _f32),
        pltpu.VMEM((EW,), _f32),
        pltpu.VMEM_SHARED((NS, 1, NP), _f32),
        pltpu.VMEM_SHARED((1, NP), _f32),
    ],
)
def _att_kernel(es_hbm, ed_hbm, src_flat, dst_flat, zrow, out,
                es_v, ed_v, src_a, dst_a, src_b, dst_b, w_v,
                hist, tmp_v, dacc_v, denom_v, att_v, stage_sh, den_sh):
    c = lax.axis_index("c")
    s = lax.axis_index("s")
    w = c * NS + s
    wm = (NC - 1 - c) * NS + s

    pltpu.sync_copy(es_hbm, es_v)
    pltpu.sync_copy(ed_hbm, ed_v)
    pltpu.sync_copy(src_flat.at[pl.ds(w * EW, EW)], src_a)
    pltpu.sync_copy(dst_flat.at[pl.ds(w * EW, EW)], dst_a)
    pltpu.sync_copy(src_flat.at[pl.ds(wm * EW, EW)], src_b)
    pltpu.sync_copy(dst_flat.at[pl.ds(wm * EW, EW)], dst_b)
    pltpu.sync_copy(zrow, hist)

    z16 = jnp.zeros((L,), jnp.int32)

    def p1a(sl):
        wv = _edge_w(es_v, ed_v, src_a[sl], dst_a[sl])
        w_v[sl] = wv
        plsc.addupdate_scatter(hist, [z16, dst_a[sl]], wv)

    def p1b(sl):
        wv = _edge_w(es_v, ed_v, src_b[sl], dst_b[sl])
        plsc.addupdate_scatter(hist, [z16, dst_b[sl]], wv)

    @pl.loop(0, EW // L - 1, step=2)
    def _(i):
        for u in range(2):
            p1a(pl.ds((i + u) * L, L))

    p1a(pl.ds(EW - L, L))

    @pl.loop(0, EW // L - 1, step=2)
    def _(i):
        for u in range(2):
            p1b(pl.ds((i + u) * L, L))

    p1b(pl.ds(EW - L, L))

    pltpu.sync_copy(hist, stage_sh.at[s])
    plsc.subcore_barrier()

    @pl.loop(0, SP // L)
    def _(i):
        dacc_v[pl.ds(i * L, L)] = jnp.zeros((L,), _f32)

    for t in range(NS):
        pltpu.sync_copy(stage_sh.at[t, 0, pl.ds(s * SP, SP)], tmp_v)

        @pl.loop(0, SP // L)
        def _(i):
            sl = pl.ds(i * L, L)
            dacc_v[sl] = dacc_v[sl] + tmp_v[sl]

    pltpu.sync_copy(dacc_v, den_sh.at[0, pl.ds(s * SP, SP)])
    plsc.subcore_barrier()
    pltpu.sync_copy(den_sh.at[0], denom_v)

    def p2(sl):
        dn = plsc.load_gather(denom_v, [dst_a[sl]])
        att_v[sl] = w_v[sl] / dn

    @pl.loop(0, EW // L - 1, step=2)
    def _(i):
        for u in range(2):
            p2(pl.ds((i + u) * L, L))

    p2(pl.ds(EW - L, L))

    pltpu.sync_copy(att_v, out.at[pl.ds(w * EW, EW)])


BLK = 2000
GRID = N // BLK
BLKP = 2048
GRIDP = NP // BLKP


def _row_spec(width, block=BLK):
    return pl.BlockSpec((block, width), lambda i: (i, 0))


def _full_spec(shape):
    return pl.BlockSpec(shape, lambda i: tuple(0 for _ in shape))


def _mlp_body(h_ref, w1_ref, b1_ref, w2_ref, b2_ref, o_ref):
    x = jnp.dot(h_ref[...], w1_ref[...], preferred_element_type=_f32)
    x = jnp.maximum(x + b1_ref[...][None, :], 0.0)
    x = jnp.dot(x, w2_ref[...], preferred_element_type=_f32)
    o_ref[...] = x + b2_ref[...][None, :]


def _mlp(h, W1, b1, W2, b2):
    return pl.pallas_call(
        _mlp_body,
        grid=(GRID,),
        in_specs=[_row_spec(D), _full_spec((D, D)), _full_spec((D,)),
                  _full_spec((D, D)), _full_spec((D,))],
        out_specs=_row_spec(D),
        out_shape=jax.ShapeDtypeStruct((NP, D), _f32),
    )(h, W1, b1, W2, b2)


def _prep_body(hs_ref, hd_ref, x_ref, a_ref, b_ref, nd_ref, fs0_ref):
    degs = jnp.sum(hs_ref[...][:, 0, :], axis=0)[:, None]
    degd = jnp.sum(hd_ref[...][:, 0, :], axis=0)[:, None]
    ns = jax.lax.rsqrt(jnp.maximum(degs, 1.0))
    nd = jax.lax.rsqrt(jnp.maximum(degd, 1.0))
    a_ref[...] = (1.0 - ALPHA) * ns * nd
    b_ref[...] = ALPHA * ns
    nd_ref[...] = nd
    fs0_ref[...] = x_ref[...] * ns


def _prep(hS, hD, x):
    return pl.pallas_call(
        _prep_body,
        grid=(GRIDP,),
        in_specs=[pl.BlockSpec((NW, 1, BLKP), lambda i: (0, 0, i)),
                  pl.BlockSpec((NW, 1, BLKP), lambda i: (0, 0, i)),
                  _row_spec(D, BLKP)],
        out_specs=(_row_spec(1, BLKP), _row_spec(1, BLKP),
                   _row_spec(1, BLKP), _row_spec(D, BLKP)),
        out_shape=(jax.ShapeDtypeStruct((NP, 1), _f32),
                   jax.ShapeDtypeStruct((NP, 1), _f32),
                   jax.ShapeDtypeStruct((NP, 1), _f32),
                   jax.ShapeDtypeStruct((NP, D), _f32)),
    )(hS, hD, x)


def _combine_body(p0, p1, x0, a_ref, b_ref, fs_ref):
    agg = p0[...][0] + p1[...][0]
    fs_ref[...] = a_ref[...] * agg + b_ref[...] * x0[...]


def _combine(acc, x0, A, B):
    return pl.pallas_call(
        _combine_body,
        grid=(GRIDP,),
        in_specs=[pl.BlockSpec((1, BLKP, D), lambda i: (0, i, 0)),
                  pl.BlockSpec((1, BLKP, D), lambda i: (1, i, 0)),
                  _row_spec(D, BLKP), _row_spec(1, BLKP),
                  _row_spec(1, BLKP)],
        out_specs=_row_spec(D, BLKP),
        out_shape=jax.ShapeDtypeStruct((NP, D), _f32),
    )(acc, acc, x0, A, B)


def _final_body(p0, p1, x0, nd, wsrc, wdst, logp_ref, es_ref, ed_ref):
    agg = p0[...][0] + p1[...][0]
    feat = (1.0 - ALPHA) * (agg * nd[...]) + ALPHA * x0[...]
    m = jnp.max(feat, axis=1, keepdims=True)
    lse = jnp.log(jnp.sum(jnp.exp(feat - m), axis=1, keepdims=True)) + m
    logp_ref[...] = feat - lse
    es_ref[...] = jnp.sum(feat * wsrc[...][:, 0][None, :], axis=1,
                          keepdims=True)
    ed_ref[...] = jnp.sum(feat * wdst[...][:, 0][None, :], axis=1,
                          keepdims=True)


def _final(acc, x0, nd, Wsrc, Wdst):
    return pl.pallas_call(
        _final_body,
        grid=(GRID,),
        in_specs=[pl.BlockSpec((1, BLK, D), lambda i: (0, i, 0)),
                  pl.BlockSpec((1, BLK, D), lambda i: (1, i, 0)),
                  _row_spec(D), _row_spec(1),
                  _full_spec((D, 1)), _full_spec((D, 1))],
        out_specs=(_row_spec(D), _row_spec(1), _row_spec(1)),
        out_shape=(jax.ShapeDtypeStruct((N, D), _f32),
                   jax.ShapeDtypeStruct((N, 1), _f32),
                   jax.ShapeDtypeStruct((N, 1), _f32)),
    )(acc, acc, x0, nd, Wsrc, Wdst)


def kernel(h, edge_index, W1, b1, W2, b2, Wsrc, Wdst):
    src = edge_index[0]
    dst = edge_index[1]
    idxm = jnp.stack([src.reshape(NW, NCHW, CPW),
                      dst.reshape(NW, NCHW, CPW)], axis=2)

    zrow = jnp.zeros((1, NP), _f32)
    zrows = jnp.zeros((SP, D), _f32)

    x = _mlp(h, W1, b1, W2, b2)
    hS, hD = _deg_kernel(src, dst, zrow)
    A, B, nd, fs0 = _prep(hS, hD, x)
    fs = fs0
    for _ in range(KHOP - 1):
        acc = _prop_kernel(fs, idxm, zrows)
        fs = _combine(acc, x, A, B)
    acc = _prop_kernel(fs, idxm, zrows)
    logp, es, ed = _final(acc, x, nd, Wsrc, Wdst)
    att = _att_kernel(es.reshape(N), ed.reshape(N), src, dst, zrow)
    return logp, att.reshape(E, 1)
